# baseline (speedup 1.0000x reference)
import functools

import jax
import jax.numpy as jnp
from jax.experimental import pallas as pl
from jax.experimental.pallas import tpu as pltpu

B, N, F = 4, 2048, 512
H1, H2, L = 512, 512, 128
_EPS = 1e-5
_NSLABS = 4
_SLAB = N // _NSLABS


def _ln_bf16(x, g, b):
    mean = jnp.mean(x, axis=-1, keepdims=True)
    xc = x - mean
    var = jnp.mean(xc * xc, axis=-1, keepdims=True)
    return (xc * jax.lax.rsqrt(var + _EPS) * g + b).astype(jnp.bfloat16)


def _bf16_dot(a_bf, b_bf):
    return jax.lax.dot_general(
        a_bf, b_bf, (((1,), (0,)), ((), ())),
        preferred_element_type=jnp.float32)


def _gcn_body(v_ref, adj_ref, g1_ref, b1_ref, w1_ref, g2_ref, b2_ref,
              w2_ref, wout_ref, bout_ref, out_ref,
              s1b_ref, s2b_ref, adjb_ref):
    r = pl.program_id(1)
    rows = pl.ds(r * _SLAB, _SLAB)

    @pl.when(r == 0)
    def _():
        xn = _ln_bf16(v_ref[0], g1_ref[...], b1_ref[...])
        w1b = w1_ref[...].astype(jnp.bfloat16)
        s1b_ref[...] = _bf16_dot(xn, w1b).astype(jnp.bfloat16)

    adjc = adj_ref[0].astype(jnp.bfloat16)
    adjb_ref[rows, :] = adjc
    h1c = jnp.maximum(_bf16_dot(adjc, s1b_ref[...]), 0.0)
    x2c = _ln_bf16(h1c, g2_ref[...], b2_ref[...])
    w2b = w2_ref[...].astype(jnp.bfloat16)
    s2b_ref[rows, :] = _bf16_dot(x2c, w2b).astype(jnp.bfloat16)

    @pl.when(r == _NSLABS - 1)
    def _():
        h2 = jnp.maximum(_bf16_dot(adjb_ref[...], s2b_ref[...]), 0.0)
        src = jnp.sum(h2, axis=-1)[None, :]
        out_ref[0] = jax.lax.dot_general(
            src, wout_ref[...], (((1,), (0,)), ((), ())),
            preferred_element_type=jnp.float32) + bout_ref[...]


@functools.partial(jax.jit, static_argnames=())
def kernel(v, adj, gamma1, beta1, W1, gamma2, beta2, W2, W_out, b_out):
    g1 = gamma1.reshape(1, F)
    b1 = beta1.reshape(1, F)
    g2 = gamma2.reshape(1, H1)
    b2 = beta2.reshape(1, H1)
    bo = b_out.reshape(1, L)

    grid = (B, _NSLABS)
    fixed_spec = lambda shape: pl.BlockSpec(shape, lambda b, r: (0,) * len(shape))

    out = pl.pallas_call(
        _gcn_body,
        grid=grid,
        in_specs=[
            pl.BlockSpec((1, N, F), lambda b, r: (b, 0, 0)),
            pl.BlockSpec((1, _SLAB, N), lambda b, r: (b, r, 0)),
            fixed_spec((1, F)),
            fixed_spec((1, F)),
            fixed_spec((F, H1)),
            fixed_spec((1, H1)),
            fixed_spec((1, H1)),
            fixed_spec((H1, H2)),
            fixed_spec((N, L)),
            fixed_spec((1, L)),
        ],
        out_specs=pl.BlockSpec((1, 1, L), lambda b, r: (b, 0, 0)),
        out_shape=jax.ShapeDtypeStruct((B, 1, L), jnp.float32),
        scratch_shapes=[pltpu.VMEM((N, H1), jnp.bfloat16),
                        pltpu.VMEM((N, H2), jnp.bfloat16),
                        pltpu.VMEM((N, N), jnp.bfloat16)],
        compiler_params=pltpu.CompilerParams(
            dimension_semantics=("arbitrary", "arbitrary"),
        ),
    )(v, adj, g1, b1, W1, g2, b2, W2, W_out, bo)
    return out.reshape(B, L)


# speedup vs baseline: 1.1226x; 1.1226x over previous
import functools

import jax
import jax.numpy as jnp
from jax.experimental import pallas as pl
from jax.experimental.pallas import tpu as pltpu

B, N, F = 4, 2048, 512
H1, H2, L = 512, 512, 128
_EPS = 1e-5
_NSLABS = 2
_SLAB = N // _NSLABS


def _ln_bf16(x, g, b):
    n = x.shape[-1]
    mean = jnp.sum(x, axis=-1, keepdims=True) * (1.0 / n)
    ex2 = jnp.sum(x * x, axis=-1, keepdims=True) * (1.0 / n)
    var = ex2 - mean * mean
    scale = jax.lax.rsqrt(var + _EPS)
    return (((x - mean) * scale) * g + b).astype(jnp.bfloat16)


def _bf16_dot(a_bf, b_bf):
    return jax.lax.dot_general(
        a_bf, b_bf, (((1,), (0,)), ((), ())),
        preferred_element_type=jnp.float32)


def _gcn_body(v_ref, adj_ref, g1_ref, b1_ref, w1_ref, g2_ref, b2_ref,
              w2_ref, wout_ref, bout_ref, out_ref,
              s1b_ref, s2b_ref, adjb_ref):
    r = pl.program_id(1)
    rows = pl.ds(r * _SLAB, _SLAB)

    @pl.when(r == 0)
    def _():
        xn = _ln_bf16(v_ref[0], g1_ref[...], b1_ref[...])
        w1b = w1_ref[...].astype(jnp.bfloat16)
        s1b_ref[...] = _bf16_dot(xn, w1b).astype(jnp.bfloat16)

    adjc = adj_ref[0].astype(jnp.bfloat16)
    adjb_ref[rows, :] = adjc
    h1c = jnp.maximum(_bf16_dot(adjc, s1b_ref[...]), 0.0)
    x2c = _ln_bf16(h1c, g2_ref[...], b2_ref[...])
    w2b = w2_ref[...].astype(jnp.bfloat16)
    s2b_ref[rows, :] = _bf16_dot(x2c, w2b).astype(jnp.bfloat16)

    @pl.when(r == _NSLABS - 1)
    def _():
        h2 = jnp.maximum(_bf16_dot(adjb_ref[...], s2b_ref[...]), 0.0)
        src = jnp.sum(h2, axis=-1)[None, :]
        out_ref[0] = jax.lax.dot_general(
            src, wout_ref[...], (((1,), (0,)), ((), ())),
            preferred_element_type=jnp.float32) + bout_ref[...]


@functools.partial(jax.jit, static_argnames=())
def kernel(v, adj, gamma1, beta1, W1, gamma2, beta2, W2, W_out, b_out):
    g1 = gamma1.reshape(1, F)
    b1 = beta1.reshape(1, F)
    g2 = gamma2.reshape(1, H1)
    b2 = beta2.reshape(1, H1)
    bo = b_out.reshape(1, L)

    grid = (B, _NSLABS)
    fixed_spec = lambda shape: pl.BlockSpec(shape, lambda b, r: (0,) * len(shape))

    out = pl.pallas_call(
        _gcn_body,
        grid=grid,
        in_specs=[
            pl.BlockSpec((1, N, F), lambda b, r: (b, 0, 0)),
            pl.BlockSpec((1, _SLAB, N), lambda b, r: (b, r, 0)),
            fixed_spec((1, F)),
            fixed_spec((1, F)),
            fixed_spec((F, H1)),
            fixed_spec((1, H1)),
            fixed_spec((1, H1)),
            fixed_spec((H1, H2)),
            fixed_spec((N, L)),
            fixed_spec((1, L)),
        ],
        out_specs=pl.BlockSpec((1, 1, L), lambda b, r: (b, 0, 0)),
        out_shape=jax.ShapeDtypeStruct((B, 1, L), jnp.float32),
        scratch_shapes=[pltpu.VMEM((N, H1), jnp.bfloat16),
                        pltpu.VMEM((N, H2), jnp.bfloat16),
                        pltpu.VMEM((N, N), jnp.bfloat16)],
        compiler_params=pltpu.CompilerParams(
            dimension_semantics=("arbitrary", "arbitrary"),
        ),
    )(v, adj, g1, b1, W1, g2, b2, W2, W_out, bo)
    return out.reshape(B, L)
